# baseline (speedup 1.0000x reference)
import functools

import jax
import jax.numpy as jnp
from jax import lax
from jax.experimental import pallas as pl
from jax.experimental.pallas import tpu as pltpu
from jax.experimental.pallas import tpu_sc as plsc

B, T, HIDDEN = 8, 576, 1024
NUM_LAYERS = 4
LAYER_DIM = HIDDEN // NUM_LAYERS
CB_SIZE = 1024
N = B * T
TILE = 768

NC, NS = 2, 16
NW = NC * NS
WPL = NW // NUM_LAYERS
TOK_W = N // WPL
CHUNK = 144
NCHUNK = TOK_W // CHUNK


def _tc_body(x_ref, cbt_ref, idx_ref, probs_ref):
    probs_acc = jnp.zeros((TILE, CB_SIZE), jnp.float32)
    iota_f = jax.lax.broadcasted_iota(
        jnp.int32, (TILE, CB_SIZE), 1).astype(jnp.float32)
    idx_cols = []
    for l in range(NUM_LAYERS):
        xb = x_ref[:, l * LAYER_DIM:(l + 1) * LAYER_DIM]
        sim = jnp.dot(xb, cbt_ref[l], preferred_element_type=jnp.float32)
        m = jnp.max(sim, axis=1, keepdims=True)
        e = jnp.exp(sim - m)
        s = jnp.sum(e, axis=1, keepdims=True)
        probs_acc = probs_acc + e * (0.25 / s)
        idxf = jnp.min(jnp.where(sim == m, iota_f, float(CB_SIZE)),
                       axis=1, keepdims=True)
        idx_cols.append(idxf.astype(jnp.int32))
    idx_ref[...] = jnp.concatenate(idx_cols, axis=1)
    probs_ref[...] = probs_acc


def _run_tc(x2d, cbt):
    out_shapes = (
        jax.ShapeDtypeStruct((N, NUM_LAYERS), jnp.int32),
        jax.ShapeDtypeStruct((N, CB_SIZE), jnp.float32),
    )
    return pl.pallas_call(
        _tc_body,
        grid=(N // TILE,),
        in_specs=[pl.BlockSpec((TILE, HIDDEN), lambda i: (i, 0)),
                  pl.BlockSpec((NUM_LAYERS, LAYER_DIM, CB_SIZE), lambda i: (0, 0, 0))],
        out_specs=(pl.BlockSpec((TILE, NUM_LAYERS), lambda i: (i, 0)),
                   pl.BlockSpec((TILE, CB_SIZE), lambda i: (i, 0))),
        out_shape=out_shapes,
    )(x2d, cbt)


def _sc_kernel(table_hbm, gidxt_hbm, quant_hbm,
               qb0_hbm, qb1_hbm, qb2_hbm, qb3_hbm, idx_v, rows_v, sem):
    qb_refs = (qb0_hbm, qb1_hbm, qb2_hbm, qb3_hbm)
    wid = lax.axis_index("s") * NC + lax.axis_index("c")
    lyr = wid // WPL
    j = wid % WPL
    for ll in range(NUM_LAYERS):
        @pl.when(lyr == ll)
        def _(ll=ll):
            t0 = j * TOK_W
            pltpu.sync_copy(gidxt_hbm.at[pl.ds(ll * N + t0, TOK_W)], idx_v)
            for c in range(NCHUNK):
                tok = t0 + c * CHUNK
                idxc = idx_v.at[pl.ds(c * CHUNK, CHUNK)]
                pltpu.async_copy(table_hbm.at[idxc], rows_v, sem).wait()
                pltpu.sync_copy(rows_v, qb_refs[ll].at[pl.ds(tok, CHUNK)])
                pltpu.sync_copy(
                    rows_v,
                    quant_hbm.at[pl.ds(tok, CHUNK),
                                 pl.ds(ll * LAYER_DIM, LAYER_DIM)])


def _run_sc(table, gidxt):
    mesh = plsc.VectorSubcoreMesh(core_axis_name="c", subcore_axis_name="s")
    out_type = (
        jax.ShapeDtypeStruct((N, HIDDEN), jnp.float32),
        jax.ShapeDtypeStruct((N, LAYER_DIM), jnp.float32),
        jax.ShapeDtypeStruct((N, LAYER_DIM), jnp.float32),
        jax.ShapeDtypeStruct((N, LAYER_DIM), jnp.float32),
        jax.ShapeDtypeStruct((N, LAYER_DIM), jnp.float32),
    )
    fn = functools.partial(
        pl.kernel, mesh=mesh,
        out_type=out_type,
        scratch_types=[
            pltpu.VMEM((TOK_W,), jnp.int32),
            pltpu.VMEM((CHUNK, LAYER_DIM), jnp.float32),
            pltpu.SemaphoreType.DMA,
        ],
    )(_sc_kernel)
    return fn(table, gidxt)


@jax.jit
def _run(x2d, cbt, table):
    idx_all, probs = _run_tc(x2d, cbt)
    gidxt = (idx_all + jnp.arange(NUM_LAYERS, dtype=jnp.int32) * CB_SIZE).T.reshape(-1)
    quant, qb0, qb1, qb2, qb3 = _run_sc(table, gidxt)
    return idx_all, probs, quant, qb0, qb1, qb2, qb3


def kernel(x, cb_topic_0, cb_topic_1, cb_style_2, cb_style_3, temperature):
    codebooks = (cb_topic_0, cb_topic_1, cb_style_2, cb_style_3)
    temp = jnp.maximum(temperature, 0.04)
    inv_t = (1.0 / temp).astype(jnp.float32)
    x2d = x.reshape(N, HIDDEN)
    cbt = jnp.stack([c.T for c in codebooks]) * inv_t
    table = jnp.concatenate(codebooks, axis=0)
    idx_all, probs, quant, qb0, qb1, qb2, qb3 = _run(x2d, cbt, table)
    quantized = quant.reshape(B, T, HIDDEN)
    indices = tuple(idx_all[:, l].reshape(B, T) for l in range(NUM_LAYERS))
    qblocks = tuple(q.reshape(B, T, LAYER_DIM) for q in (qb0, qb1, qb2, qb3))
    avg_code_probs = probs.reshape(B, T, CB_SIZE)
    return (quantized, indices, qblocks, avg_code_probs, x)


# speedup vs baseline: 1.3270x; 1.3270x over previous
import jax
import jax.numpy as jnp
from jax.experimental import pallas as pl
from jax.experimental.pallas import tpu as pltpu

B, T, HIDDEN = 8, 576, 1024
NUM_LAYERS = 4
LAYER_DIM = HIDDEN // NUM_LAYERS
CB_SIZE = 1024
N = B * T
TILE = 768


def _body(x_ref, cbt_ref, cb0_ref, cb1_ref, cb2_ref, cb3_ref,
          quant_ref, qb0_ref, qb1_ref, qb2_ref, qb3_ref, idx_ref, probs_ref):
    qb_refs = (qb0_ref, qb1_ref, qb2_ref, qb3_ref)
    cb_refs = (cb0_ref, cb1_ref, cb2_ref, cb3_ref)
    probs_acc = None
    iota_f = jax.lax.broadcasted_iota(
        jnp.int32, (TILE, CB_SIZE), 1).astype(jnp.float32)
    idx_cols = []
    for l in range(NUM_LAYERS):
        xb = x_ref[:, l * LAYER_DIM:(l + 1) * LAYER_DIM]
        sim = jnp.dot(xb, cbt_ref[l], preferred_element_type=jnp.float32)
        m = jnp.max(sim, axis=1, keepdims=True)
        e = jnp.exp(sim - m)
        s = jnp.sum(e, axis=1, keepdims=True)
        p = e * (0.25 / s)
        probs_acc = p if probs_acc is None else probs_acc + p
        idxf = jnp.min(jnp.where(sim == m, iota_f, float(CB_SIZE)),
                       axis=1, keepdims=True)
        idx_cols.append(idxf.astype(jnp.int32))
        onehot = (iota_f == idxf).astype(jnp.float32)
        hard = jnp.dot(onehot, cb_refs[l][...],
                       preferred_element_type=jnp.float32)
        qb_refs[l][...] = hard
        quant_ref[:, l * LAYER_DIM:(l + 1) * LAYER_DIM] = hard
    idx_ref[...] = jnp.concatenate(idx_cols, axis=1)
    probs_ref[...] = probs_acc


@jax.jit
def _run(x2d, cbt, cb0, cb1, cb2, cb3):
    out_shapes = (
        jax.ShapeDtypeStruct((N, HIDDEN), jnp.float32),
        jax.ShapeDtypeStruct((N, LAYER_DIM), jnp.float32),
        jax.ShapeDtypeStruct((N, LAYER_DIM), jnp.float32),
        jax.ShapeDtypeStruct((N, LAYER_DIM), jnp.float32),
        jax.ShapeDtypeStruct((N, LAYER_DIM), jnp.float32),
        jax.ShapeDtypeStruct((N, NUM_LAYERS), jnp.int32),
        jax.ShapeDtypeStruct((N, CB_SIZE), jnp.float32),
    )
    grid = (N // TILE,)
    out_specs = (
        pl.BlockSpec((TILE, HIDDEN), lambda i: (i, 0)),
        pl.BlockSpec((TILE, LAYER_DIM), lambda i: (i, 0)),
        pl.BlockSpec((TILE, LAYER_DIM), lambda i: (i, 0)),
        pl.BlockSpec((TILE, LAYER_DIM), lambda i: (i, 0)),
        pl.BlockSpec((TILE, LAYER_DIM), lambda i: (i, 0)),
        pl.BlockSpec((TILE, NUM_LAYERS), lambda i: (i, 0)),
        pl.BlockSpec((TILE, CB_SIZE), lambda i: (i, 0)),
    )
    return pl.pallas_call(
        _body,
        grid=grid,
        in_specs=[pl.BlockSpec((TILE, HIDDEN), lambda i: (i, 0)),
                  pl.BlockSpec((NUM_LAYERS, LAYER_DIM, CB_SIZE), lambda i: (0, 0, 0)),
                  pl.BlockSpec((CB_SIZE, LAYER_DIM), lambda i: (0, 0)),
                  pl.BlockSpec((CB_SIZE, LAYER_DIM), lambda i: (0, 0)),
                  pl.BlockSpec((CB_SIZE, LAYER_DIM), lambda i: (0, 0)),
                  pl.BlockSpec((CB_SIZE, LAYER_DIM), lambda i: (0, 0))],
        out_specs=out_specs,
        out_shape=out_shapes,
    )(x2d, cbt, cb0, cb1, cb2, cb3)


def kernel(x, cb_topic_0, cb_topic_1, cb_style_2, cb_style_3, temperature):
    codebooks = (cb_topic_0, cb_topic_1, cb_style_2, cb_style_3)
    temp = jnp.maximum(temperature, 0.04)
    inv_t = (1.0 / temp).astype(jnp.float32)
    x2d = x.reshape(N, HIDDEN)
    cbt = jnp.stack([c.T for c in codebooks]) * inv_t
    quant, qb0, qb1, qb2, qb3, idx_all, probs = _run(x2d, cbt, *codebooks)
    quantized = quant.reshape(B, T, HIDDEN)
    indices = tuple(idx_all[:, l].reshape(B, T) for l in range(NUM_LAYERS))
    qblocks = tuple(q.reshape(B, T, LAYER_DIM) for q in (qb0, qb1, qb2, qb3))
    avg_code_probs = probs.reshape(B, T, CB_SIZE)
    return (quantized, indices, qblocks, avg_code_probs, x)


# speedup vs baseline: 1.3407x; 1.0103x over previous
import jax
import jax.numpy as jnp
from jax.experimental import pallas as pl
from jax.experimental.pallas import tpu as pltpu

B, T, HIDDEN = 8, 576, 1024
NUM_LAYERS = 4
LAYER_DIM = HIDDEN // NUM_LAYERS
CB_SIZE = 1024
N = B * T
TILE = 576


def _body(x_ref, cbt_ref, cb0_ref, cb1_ref, cb2_ref, cb3_ref,
          quant_ref, qb0_ref, qb1_ref, qb2_ref, qb3_ref, idx_ref, probs_ref):
    qb_refs = (qb0_ref, qb1_ref, qb2_ref, qb3_ref)
    cb_refs = (cb0_ref, cb1_ref, cb2_ref, cb3_ref)
    probs_acc = None
    iota_f = jax.lax.broadcasted_iota(
        jnp.int32, (TILE, CB_SIZE), 1).astype(jnp.float32)
    idx_cols = []
    for l in range(NUM_LAYERS):
        xb = x_ref[:, l * LAYER_DIM:(l + 1) * LAYER_DIM]
        sim = jnp.dot(xb, cbt_ref[l], preferred_element_type=jnp.float32)
        m = jnp.max(sim, axis=1, keepdims=True)
        e = jnp.exp(sim - m)
        s = jnp.sum(e, axis=1, keepdims=True)
        p = e * (0.25 / s)
        probs_acc = p if probs_acc is None else probs_acc + p
        idxf = jnp.min(jnp.where(sim == m, iota_f, float(CB_SIZE)),
                       axis=1, keepdims=True)
        idx_cols.append(idxf.astype(jnp.int32))
        onehot = (iota_f == idxf).astype(jnp.float32)
        hard = jnp.dot(onehot, cb_refs[l][...],
                       preferred_element_type=jnp.float32)
        qb_refs[l][...] = hard
        quant_ref[:, l * LAYER_DIM:(l + 1) * LAYER_DIM] = hard
    idx_ref[...] = jnp.concatenate(idx_cols, axis=1)
    probs_ref[...] = probs_acc


@jax.jit
def _run(x2d, cbt, cb0, cb1, cb2, cb3):
    out_shapes = (
        jax.ShapeDtypeStruct((N, HIDDEN), jnp.float32),
        jax.ShapeDtypeStruct((N, LAYER_DIM), jnp.float32),
        jax.ShapeDtypeStruct((N, LAYER_DIM), jnp.float32),
        jax.ShapeDtypeStruct((N, LAYER_DIM), jnp.float32),
        jax.ShapeDtypeStruct((N, LAYER_DIM), jnp.float32),
        jax.ShapeDtypeStruct((N, NUM_LAYERS), jnp.int32),
        jax.ShapeDtypeStruct((N, CB_SIZE), jnp.float32),
    )
    grid = (N // TILE,)
    out_specs = (
        pl.BlockSpec((TILE, HIDDEN), lambda i: (i, 0)),
        pl.BlockSpec((TILE, LAYER_DIM), lambda i: (i, 0)),
        pl.BlockSpec((TILE, LAYER_DIM), lambda i: (i, 0)),
        pl.BlockSpec((TILE, LAYER_DIM), lambda i: (i, 0)),
        pl.BlockSpec((TILE, LAYER_DIM), lambda i: (i, 0)),
        pl.BlockSpec((TILE, NUM_LAYERS), lambda i: (i, 0)),
        pl.BlockSpec((TILE, CB_SIZE), lambda i: (i, 0)),
    )
    return pl.pallas_call(
        _body,
        grid=grid,
        in_specs=[pl.BlockSpec((TILE, HIDDEN), lambda i: (i, 0)),
                  pl.BlockSpec((NUM_LAYERS, LAYER_DIM, CB_SIZE), lambda i: (0, 0, 0)),
                  pl.BlockSpec((CB_SIZE, LAYER_DIM), lambda i: (0, 0)),
                  pl.BlockSpec((CB_SIZE, LAYER_DIM), lambda i: (0, 0)),
                  pl.BlockSpec((CB_SIZE, LAYER_DIM), lambda i: (0, 0)),
                  pl.BlockSpec((CB_SIZE, LAYER_DIM), lambda i: (0, 0))],
        out_specs=out_specs,
        out_shape=out_shapes,
    )(x2d, cbt, cb0, cb1, cb2, cb3)


def kernel(x, cb_topic_0, cb_topic_1, cb_style_2, cb_style_3, temperature):
    codebooks = (cb_topic_0, cb_topic_1, cb_style_2, cb_style_3)
    temp = jnp.maximum(temperature, 0.04)
    inv_t = (1.0 / temp).astype(jnp.float32)
    x2d = x.reshape(N, HIDDEN)
    cbt = jnp.stack([c.T for c in codebooks]) * inv_t
    quant, qb0, qb1, qb2, qb3, idx_all, probs = _run(x2d, cbt, *codebooks)
    quantized = quant.reshape(B, T, HIDDEN)
    indices = tuple(idx_all[:, l].reshape(B, T) for l in range(NUM_LAYERS))
    qblocks = tuple(q.reshape(B, T, LAYER_DIM) for q in (qb0, qb1, qb2, qb3))
    avg_code_probs = probs.reshape(B, T, CB_SIZE)
    return (quantized, indices, qblocks, avg_code_probs, x)
